# baseline (speedup 1.0000x reference)
import functools

import jax
import jax.numpy as jnp
from jax import lax
from jax.experimental import pallas as pl
from jax.experimental.pallas import tpu as pltpu
from jax.experimental.pallas import tpu_sc as plsc

_NC = 2
_NS = 16
_NW = _NC * _NS
_LANES = 16

_K = 80
_RPAD = 480
_BPAD = 64
_APAD = 8
_HH = 64
_HE = 80


def _prep_tables_kernel(re_ref, wr_ref, req_ref, wq_ref, bq_ref,
                        ra_ref, rb_ref, aq_ref):
    Ln = re_ref.shape[0]
    R = re_ref.shape[1]
    z = jnp.zeros((R, _APAD), jnp.float32)
    for i in range(Ln):
        re_i = re_ref[i]
        ar = lax.dot_general(re_i, wr_ref[i], (((1,), (1,)), ((), ())),
                             preferred_element_type=jnp.float32)
        ra_ref[i] = jnp.concatenate([re_i[:, :_HH], ar, z], axis=1)
        rb_ref[i] = jnp.concatenate([re_i[:, _HH:], ar, z], axis=1)
        aq_ref[i] = lax.dot_general(
            wq_ref[i], req_ref[i], (((1,), (1,)), ((), ())),
            preferred_element_type=jnp.float32) + bq_ref[i]


def _hext0_kernel(h_ref, ws_ref, outa_ref, outb_ref):
    h = h_ref[...]
    at = lax.dot_general(h, ws_ref[...], (((1,), (1,)), ((), ())),
                         preferred_element_type=jnp.float32)
    z = jnp.zeros((h.shape[0], _APAD), jnp.float32)
    outa_ref[...] = jnp.concatenate([h[:, :_HH], at, z], axis=1)
    outb_ref[...] = jnp.concatenate([h[:, _HH:], at, z], axis=1)


def _dense_layer_kernel(a0_ref, a1_ref, b0_ref, b1_ref, h0_ref, wh_ref,
                        wih_ref, whh_ref, bih_ref, bhh_ref, wn_ref,
                        outa_ref, outb_ref):
    agg = jnp.concatenate([a0_ref[...] + a1_ref[...],
                           b0_ref[...] + b1_ref[...]], axis=1)
    hid = jnp.maximum(
        lax.dot_general(agg, wh_ref[...], (((1,), (1,)), ((), ())),
                        preferred_element_type=jnp.float32), 0.0)
    act0 = (jnp.sum(hid, axis=1, keepdims=True) == 0.0)
    mask = jnp.where(act0, 0.0, 1.0)
    Hd = hid.shape[1]
    h0 = jnp.concatenate([h0_ref[:, :_HH], h0_ref[:, _HE:_HE + _HH]], axis=1)
    gi = lax.dot_general(hid, wih_ref[...], (((1,), (1,)), ((), ())),
                         preferred_element_type=jnp.float32) + bih_ref[0:1, :]
    gh = lax.dot_general(h0, whh_ref[...], (((1,), (1,)), ((), ())),
                         preferred_element_type=jnp.float32) + bhh_ref[0:1, :]
    i_r, i_z, i_n = gi[:, :Hd], gi[:, Hd:2 * Hd], gi[:, 2 * Hd:]
    h_r, h_z, h_n = gh[:, :Hd], gh[:, Hd:2 * Hd], gh[:, 2 * Hd:]
    r = jax.nn.sigmoid(i_r + h_r)
    z = jax.nn.sigmoid(i_z + h_z)
    n = jnp.tanh(i_n + r * h_n)
    hnew = (1.0 - z) * n + z * h0
    st = hnew * mask
    at = lax.dot_general(st, wn_ref[...], (((1,), (1,)), ((), ())),
                         preferred_element_type=jnp.float32)
    zp = jnp.zeros((st.shape[0], _APAD), jnp.float32)
    outa_ref[...] = jnp.concatenate([st[:, :_HH], at, zp], axis=1)
    outb_ref[...] = jnp.concatenate([st[:, _HH:], at, zp], axis=1)


def _make_sc_edge(NP, E):
    EW = E // _NW
    NCHUNK = EW // _K
    NR = NP // _NS
    RC = _K
    NRC = NR // RC
    AOF = _HH
    mesh = plsc.VectorSubcoreMesh(core_axis_name="c", subcore_axis_name="s",
                                  num_cores=_NC, num_subcores=_NS)

    @functools.partial(
        pl.kernel,
        out_type=jax.ShapeDtypeStruct((_NC * NP, _HH), jnp.float32),
        mesh=mesh,
        compiler_params=pltpu.CompilerParams(use_tc_tiling_on_sc=False,
                                             needs_layout_passes=False),
        scratch_types=[
            pltpu.VMEM_SHARED((NP, _HH), jnp.float32),
            pltpu.VMEM((5 * _BPAD,), jnp.float32),
            pltpu.VMEM((_LANES,), jnp.float32),
            pltpu.VMEM((2, 4, _K), jnp.int32),
            pltpu.VMEM((2, _K, _HE), jnp.float32),
            pltpu.VMEM((2, _K, _HE), jnp.float32),
            pltpu.VMEM((2, _K, _HH), jnp.float32),
            pltpu.VMEM((2, _K), jnp.int32),
            pltpu.VMEM((_K,), jnp.float32),
            pltpu.SemaphoreType.DMA,
            pltpu.SemaphoreType.DMA,
            pltpu.SemaphoreType.DMA,
            pltpu.SemaphoreType.DMA,
            pltpu.SemaphoreType.DMA,
            pltpu.SemaphoreType.DMA,
        ],
    )
    def sc_edge(hext_hbm, rext_hbm, aq_hbm, wal_hbm,
                idx_hbm, out_hbm,
                agg_sh, cols_q, wbuf,
                idx_c, hext_buf, rext_buf, msg_buf, obj_g, alpha_buf,
                semh0, semh1, semr0, semr1, sems0, sems1):
        cid = lax.axis_index("c")
        sid = lax.axis_index("s")
        wid = sid * _NC + cid
        semh = (semh0, semh1)
        semr = (semr0, semr1)
        sems = (sems0, sems1)

        for a in range(5):
            pltpu.sync_copy(aq_hbm.at[a], cols_q.at[pl.ds(a * _BPAD, _BPAD)])
        pltpu.sync_copy(wal_hbm, wbuf)

        bounce = msg_buf.at[0]
        def zrow(rr, _):
            for j in range(_HH // _LANES):
                bounce[rr, pl.ds(_LANES * j, _LANES)] = jnp.zeros(
                    (_LANES,), jnp.float32)
            return 0
        lax.fori_loop(0, RC, zrow, 0)
        for t in range(NRC):
            pltpu.sync_copy(bounce, agg_sh.at[pl.ds(sid * NR + t * RC, RC)])
        plsc.subcore_barrier()

        blk0 = wid * NCHUNK
        wv = wbuf[...]
        lanes = jnp.arange(_LANES, dtype=jnp.int32)

        def fetch(blk, s):
            pltpu.sync_copy(idx_hbm.at[blk], idx_c.at[s])
            pltpu.async_copy(hext_hbm.at[idx_c.at[s].at[0]],
                             hext_buf.at[s], semh[s])
            pltpu.async_copy(rext_hbm.at[idx_c.at[s].at[1]],
                             rext_buf.at[s], semr[s])

        def compute(blk, s, wait_pred, fetch_pred):
            hb = hext_buf.at[s]
            rb = rext_buf.at[s]
            ms = msg_buf.at[s]
            og = obj_g.at[s]
            ic = idx_c.at[s]
            pltpu.make_async_copy(hext_hbm.at[ic.at[0]], hb, semh[s]).wait()
            pltpu.make_async_copy(rext_hbm.at[ic.at[1]], rb, semr[s]).wait()

            for g in range(_K // _LANES):
                sl = pl.ds(_LANES * g, _LANES)
                og[sl] = ic[2, sl]

            for g in range(_K // _LANES):
                sl = pl.ds(_LANES * g, _LANES)
                e16 = lanes + (_LANES * g)
                eb16 = ic[3, sl]
                acc = jnp.full((_LANES,), wv[5])
                for a in range(5):
                    ca = jnp.full((_LANES,), AOF + a, jnp.int32)
                    va = (plsc.load_gather(hb, [e16, ca])
                          + plsc.load_gather(rb, [e16, ca])
                          + plsc.load_gather(cols_q, [eb16 + a * _BPAD]))
                    va = jnp.maximum(va, 0.0)
                    acc = acc + jnp.full((_LANES,), wv[a]) * va
                alpha_buf[sl] = 1.0 / (1.0 + jnp.exp(-acc))

            @pl.when(wait_pred)
            def _():
                pltpu.make_async_copy(ms, agg_sh.at[og], sems[s]).wait()

            @plsc.parallel_loop(0, _K, step=1, unroll=16)
            def _(e):
                av = plsc.load_gather(
                    alpha_buf, [jnp.full((_LANES,), e, jnp.int32)])
                for j in range(_HH // _LANES):
                    s2 = pl.ds(_LANES * j, _LANES)
                    ms[e, s2] = hb[e, s2] * rb[e, s2] * av

            pltpu.async_copy(ms, agg_sh.at[og], sems[s], add=True)

            @pl.when(fetch_pred)
            def _():
                fetch(blk + 2, s)

        false_p = jnp.bool_(False)
        true_p = jnp.bool_(True)

        fetch(blk0, 0)
        fetch(blk0 + 1, 1)

        def pair_body(p, _):
            c0 = blk0 + 2 * p
            compute(c0, 0, p > 0, true_p)
            compute(c0 + 1, 1, p > 0, (2 * p + 3) < NCHUNK)
            return 0
        lax.fori_loop(0, (NCHUNK - 1) // 2, pair_body, 0)
        compute(blk0 + NCHUNK - 1, 0, true_p, false_p)

        pltpu.make_async_copy(msg_buf.at[0], agg_sh.at[obj_g.at[0]],
                              sems0).wait()
        pltpu.make_async_copy(msg_buf.at[1], agg_sh.at[obj_g.at[1]],
                              sems1).wait()
        plsc.subcore_barrier()

        for t in range(NRC):
            row = sid * NR + t * RC
            pltpu.sync_copy(agg_sh.at[pl.ds(row, RC)], bounce)
            pltpu.sync_copy(bounce, out_hbm.at[pl.ds(cid * NP + row, RC)])

    return sc_edge


def kernel(q_sub, q_rel, batch_idxs, query_sub_idxs, edge_batch_idxs, edges,
           rela_embed, Ws, Wr, Wqr_w, Wqr_b, walpha_w, walpha_b, Wh,
           gru_w_ih, gru_w_hh, gru_b_ih, gru_b_hh, W_final):
    N = batch_idxs.shape[0]
    Ln, R, H = rela_embed.shape
    A = Ws.shape[1]
    E = edges.shape[0]
    MB = 512
    NP = 10240
    NBLK = NP // MB

    sub = edges[:, 0].astype(jnp.int32)
    rel = edges[:, 1].astype(jnp.int32)
    obj = edges[:, 2].astype(jnp.int32)
    eb = edge_batch_idxs.astype(jnp.int32)
    EW = E // _NW
    NCHUNK = EW // _K
    idx_pack = (jnp.stack([sub, rel, obj, eb], 0)
                .reshape(4, _NW, NCHUNK, _K)
                .transpose(1, 2, 0, 3)
                .reshape(_NW * NCHUNK, 4, _K))

    h = jnp.zeros((NP, H), jnp.float32).at[query_sub_idxs].set(1.0)
    h0ext = jnp.zeros((NP, 2 * _HE), jnp.float32)

    rela_pad = jnp.pad(rela_embed, ((0, 0), (0, _RPAD - R), (0, 0)))
    re_q = jnp.pad(rela_embed[:, q_rel, :],
                   ((0, 0), (0, _BPAD - q_rel.shape[0]), (0, 0)))
    Ws_p = jnp.pad(Ws, ((0, 0), (0, _APAD - A), (0, 0)))
    Wr_p = jnp.pad(Wr, ((0, 0), (0, _APAD - A), (0, 0)))
    Wq_p = jnp.pad(Wqr_w, ((0, 0), (0, _APAD - A), (0, 0)))
    Wf_p = jnp.pad(W_final, ((0, _APAD - 1), (0, 0)))
    bq = jnp.broadcast_to(
        jnp.pad(Wqr_b, ((0, 0), (0, _APAD - A)))[:, :, None], (Ln, _APAD, _BPAD))
    wal = jnp.concatenate(
        [walpha_w[:, 0, :], walpha_b,
         jnp.zeros((Ln, _LANES - A - 1), jnp.float32)], axis=1)
    bih2 = jnp.broadcast_to(gru_b_ih[None, :], (8, 3 * H))
    bhh2 = jnp.broadcast_to(gru_b_hh[None, :], (8, 3 * H))

    rextA, rextB, aqT = pl.pallas_call(
        _prep_tables_kernel,
        out_shape=[jax.ShapeDtypeStruct((Ln, _RPAD, _HE), jnp.float32),
                   jax.ShapeDtypeStruct((Ln, _RPAD, _HE), jnp.float32),
                   jax.ShapeDtypeStruct((Ln, _APAD, _BPAD), jnp.float32)],
    )(rela_pad, Wr_p, re_q, Wq_p, bq)

    hextA, hextB = pl.pallas_call(
        _hext0_kernel,
        grid=(NBLK,),
        in_specs=[pl.BlockSpec((MB, H), lambda j: (j, 0)),
                  pl.BlockSpec((_APAD, H), lambda j: (0, 0))],
        out_specs=[pl.BlockSpec((MB, _HE), lambda j: (j, 0)),
                   pl.BlockSpec((MB, _HE), lambda j: (j, 0))],
        out_shape=[jax.ShapeDtypeStruct((NP, _HE), jnp.float32),
                   jax.ShapeDtypeStruct((NP, _HE), jnp.float32)],
    )(h, Ws_p[0])

    sc_edge = _make_sc_edge(NP, E)

    dense_call = pl.pallas_call(
        _dense_layer_kernel,
        grid=(NBLK,),
        in_specs=[pl.BlockSpec((MB, _HH), lambda j: (j, 0)),
                  pl.BlockSpec((MB, _HH), lambda j: (j + NBLK, 0)),
                  pl.BlockSpec((MB, _HH), lambda j: (j, 0)),
                  pl.BlockSpec((MB, _HH), lambda j: (j + NBLK, 0)),
                  pl.BlockSpec((MB, 2 * _HE), lambda j: (j, 0)),
                  pl.BlockSpec((H, H), lambda j: (0, 0)),
                  pl.BlockSpec((3 * H, H), lambda j: (0, 0)),
                  pl.BlockSpec((3 * H, H), lambda j: (0, 0)),
                  pl.BlockSpec((8, 3 * H), lambda j: (0, 0)),
                  pl.BlockSpec((8, 3 * H), lambda j: (0, 0)),
                  pl.BlockSpec((_APAD, H), lambda j: (0, 0))],
        out_specs=[pl.BlockSpec((MB, _HE), lambda j: (j, 0)),
                   pl.BlockSpec((MB, _HE), lambda j: (j, 0))],
        out_shape=[jax.ShapeDtypeStruct((NP, _HE), jnp.float32),
                   jax.ShapeDtypeStruct((NP, _HE), jnp.float32)],
    )

    for i in range(Ln):
        aggA = sc_edge(hextA, rextA[i], aqT[i], wal[i], idx_pack)
        aggB = sc_edge(hextB, rextB[i], aqT[i], wal[i], idx_pack)
        wnext = Ws_p[i + 1] if i + 1 < Ln else Wf_p
        hextA, hextB = dense_call(aggA, aggA, aggB, aggB, h0ext, Wh[i],
                                  gru_w_ih, gru_w_hh, bih2, bhh2, wnext)
        h0ext = jnp.concatenate([hextA, hextB], axis=1)

    return hextA[:N, _HH]


# speedup vs baseline: 1.0215x; 1.0215x over previous
import functools

import jax
import jax.numpy as jnp
from jax import lax
from jax.experimental import pallas as pl
from jax.experimental.pallas import tpu as pltpu
from jax.experimental.pallas import tpu_sc as plsc

_NC = 2
_NS = 16
_NW = _NC * _NS
_LANES = 16

_K = 80
_RPAD = 480
_BPAD = 64
_APAD = 8
_HH = 64
_HE = 80


def _prep_tables_kernel(re_ref, wr_ref, req_ref, wq_ref, bq_ref,
                        ra_ref, rb_ref, aq_ref):
    Ln = re_ref.shape[0]
    R = re_ref.shape[1]
    z = jnp.zeros((R, _APAD), jnp.float32)
    for i in range(Ln):
        re_i = re_ref[i]
        ar = lax.dot_general(re_i, wr_ref[i], (((1,), (1,)), ((), ())),
                             preferred_element_type=jnp.float32)
        ra_ref[i] = jnp.concatenate([re_i[:, :_HH], ar, z], axis=1)
        rb_ref[i] = jnp.concatenate([re_i[:, _HH:], ar, z], axis=1)
        aq_ref[i] = lax.dot_general(
            wq_ref[i], req_ref[i], (((1,), (1,)), ((), ())),
            preferred_element_type=jnp.float32) + bq_ref[i]


def _hext0_kernel(h_ref, ws_ref, outa_ref, outb_ref):
    h = h_ref[...]
    at = lax.dot_general(h, ws_ref[...], (((1,), (1,)), ((), ())),
                         preferred_element_type=jnp.float32)
    z = jnp.zeros((h.shape[0], _APAD), jnp.float32)
    outa_ref[...] = jnp.concatenate([h[:, :_HH], at, z], axis=1)
    outb_ref[...] = jnp.concatenate([h[:, _HH:], at, z], axis=1)


def _dense_layer_kernel(a0_ref, a1_ref, b0_ref, b1_ref, h0_ref, wh_ref,
                        wih_ref, whh_ref, bih_ref, bhh_ref, wn_ref,
                        outa_ref, outb_ref):
    agg = jnp.concatenate([a0_ref[...] + a1_ref[...],
                           b0_ref[...] + b1_ref[...]], axis=1)
    hid = jnp.maximum(
        lax.dot_general(agg, wh_ref[...], (((1,), (1,)), ((), ())),
                        preferred_element_type=jnp.float32), 0.0)
    act0 = (jnp.sum(hid, axis=1, keepdims=True) == 0.0)
    mask = jnp.where(act0, 0.0, 1.0)
    Hd = hid.shape[1]
    h0 = jnp.concatenate([h0_ref[:, :_HH], h0_ref[:, _HE:_HE + _HH]], axis=1)
    gi = lax.dot_general(hid, wih_ref[...], (((1,), (1,)), ((), ())),
                         preferred_element_type=jnp.float32) + bih_ref[0:1, :]
    gh = lax.dot_general(h0, whh_ref[...], (((1,), (1,)), ((), ())),
                         preferred_element_type=jnp.float32) + bhh_ref[0:1, :]
    i_r, i_z, i_n = gi[:, :Hd], gi[:, Hd:2 * Hd], gi[:, 2 * Hd:]
    h_r, h_z, h_n = gh[:, :Hd], gh[:, Hd:2 * Hd], gh[:, 2 * Hd:]
    r = jax.nn.sigmoid(i_r + h_r)
    z = jax.nn.sigmoid(i_z + h_z)
    n = jnp.tanh(i_n + r * h_n)
    hnew = (1.0 - z) * n + z * h0
    st = hnew * mask
    at = lax.dot_general(st, wn_ref[...], (((1,), (1,)), ((), ())),
                         preferred_element_type=jnp.float32)
    zp = jnp.zeros((st.shape[0], _APAD), jnp.float32)
    outa_ref[...] = jnp.concatenate([st[:, :_HH], at, zp], axis=1)
    outb_ref[...] = jnp.concatenate([st[:, _HH:], at, zp], axis=1)


def _make_sc_edge(NP, E):
    EW = E // _NW
    NCHUNK = EW // _K
    NR = NP // _NS
    RC = _K
    NRC = NR // RC
    AOF = _HH
    mesh = plsc.VectorSubcoreMesh(core_axis_name="c", subcore_axis_name="s",
                                  num_cores=_NC, num_subcores=_NS)

    @functools.partial(
        pl.kernel,
        out_type=jax.ShapeDtypeStruct((_NC * NP, _HH), jnp.float32),
        mesh=mesh,
        compiler_params=pltpu.CompilerParams(use_tc_tiling_on_sc=False,
                                             needs_layout_passes=False),
        scratch_types=[
            pltpu.VMEM_SHARED((NP, _HH), jnp.float32),
            pltpu.VMEM((5 * _BPAD,), jnp.float32),
            pltpu.VMEM((_LANES,), jnp.float32),
            pltpu.VMEM((2, 4, _K), jnp.int32),
            pltpu.VMEM((2, _K, _HE), jnp.float32),
            pltpu.VMEM((2, _K, _HE), jnp.float32),
            pltpu.VMEM((2, _K, _HH), jnp.float32),
            pltpu.VMEM((2, _K), jnp.int32),
            pltpu.VMEM((_K,), jnp.float32),
            pltpu.SemaphoreType.DMA,
            pltpu.SemaphoreType.DMA,
            pltpu.SemaphoreType.DMA,
            pltpu.SemaphoreType.DMA,
            pltpu.SemaphoreType.DMA,
            pltpu.SemaphoreType.DMA,
        ],
    )
    def sc_edge(hext_hbm, rext_hbm, aq_hbm, wal_hbm,
                idx_hbm, out_hbm,
                agg_sh, cols_q, wbuf,
                idx_c, hext_buf, rext_buf, msg_buf, obj_g, alpha_buf,
                semh0, semh1, semr0, semr1, sems0, sems1):
        cid = lax.axis_index("c")
        sid = lax.axis_index("s")
        wid = sid * _NC + cid
        semh = (semh0, semh1)
        semr = (semr0, semr1)
        sems = (sems0, sems1)

        for a in range(5):
            pltpu.sync_copy(aq_hbm.at[a], cols_q.at[pl.ds(a * _BPAD, _BPAD)])
        pltpu.sync_copy(wal_hbm, wbuf)

        bounce = msg_buf.at[0]
        def zrow(rr, _):
            for j in range(_HH // _LANES):
                bounce[rr, pl.ds(_LANES * j, _LANES)] = jnp.zeros(
                    (_LANES,), jnp.float32)
            return 0
        lax.fori_loop(0, RC, zrow, 0)
        for t in range(NRC):
            pltpu.sync_copy(bounce, agg_sh.at[pl.ds(sid * NR + t * RC, RC)])
        plsc.subcore_barrier()

        blk0 = wid * NCHUNK
        wv = wbuf[...]
        lanes = jnp.arange(_LANES, dtype=jnp.int32)

        def fetch(blk, s):
            pltpu.sync_copy(idx_hbm.at[blk], idx_c.at[s])
            pltpu.async_copy(hext_hbm.at[idx_c.at[s].at[0]],
                             hext_buf.at[s], semh[s])
            pltpu.async_copy(rext_hbm.at[idx_c.at[s].at[1]],
                             rext_buf.at[s], semr[s])

        def compute(blk, s, wait_pred, fetch_pred):
            hb = hext_buf.at[s]
            rb = rext_buf.at[s]
            ms = msg_buf.at[s]
            og = obj_g.at[s]
            ic = idx_c.at[s]
            pltpu.make_async_copy(hext_hbm.at[ic.at[0]], hb, semh[s]).wait()
            pltpu.make_async_copy(rext_hbm.at[ic.at[1]], rb, semr[s]).wait()

            @plsc.parallel_loop(0, _K // _LANES, step=1,
                                unroll=_K // _LANES)
            def _(g):
                sl = pl.ds(_LANES * g, _LANES)
                e16 = lanes + (_LANES * g)
                og[sl] = ic[2, sl]
                eb16 = ic[3, sl]
                acc = jnp.full((_LANES,), wv[5])
                for a in range(5):
                    ca = jnp.full((_LANES,), AOF + a, jnp.int32)
                    va = (plsc.load_gather(hb, [e16, ca])
                          + plsc.load_gather(rb, [e16, ca])
                          + plsc.load_gather(cols_q, [eb16 + a * _BPAD]))
                    va = jnp.maximum(va, 0.0)
                    acc = acc + jnp.full((_LANES,), wv[a]) * va
                alpha_buf[sl] = 1.0 / (1.0 + jnp.exp(-acc))

            @pl.when(wait_pred)
            def _():
                pltpu.make_async_copy(ms, agg_sh.at[og], sems[s]).wait()

            @plsc.parallel_loop(0, _K, step=1, unroll=8)
            def _(e):
                av = plsc.load_gather(
                    alpha_buf, [jnp.full((_LANES,), e, jnp.int32)])
                for j in range(_HH // _LANES):
                    s2 = pl.ds(_LANES * j, _LANES)
                    ms[e, s2] = hb[e, s2] * rb[e, s2] * av

            pltpu.async_copy(ms, agg_sh.at[og], sems[s], add=True)

            @pl.when(fetch_pred)
            def _():
                fetch(blk + 2, s)

        false_p = jnp.bool_(False)
        true_p = jnp.bool_(True)

        fetch(blk0, 0)
        fetch(blk0 + 1, 1)

        def pair_body(p, _):
            c0 = blk0 + 2 * p
            compute(c0, 0, p > 0, true_p)
            compute(c0 + 1, 1, p > 0, (2 * p + 3) < NCHUNK)
            return 0
        lax.fori_loop(0, (NCHUNK - 1) // 2, pair_body, 0)
        compute(blk0 + NCHUNK - 1, 0, true_p, false_p)

        pltpu.make_async_copy(msg_buf.at[0], agg_sh.at[obj_g.at[0]],
                              sems0).wait()
        pltpu.make_async_copy(msg_buf.at[1], agg_sh.at[obj_g.at[1]],
                              sems1).wait()
        plsc.subcore_barrier()

        for t in range(NRC):
            row = sid * NR + t * RC
            pltpu.sync_copy(agg_sh.at[pl.ds(row, RC)], bounce)
            pltpu.sync_copy(bounce, out_hbm.at[pl.ds(cid * NP + row, RC)])

    return sc_edge


def kernel(q_sub, q_rel, batch_idxs, query_sub_idxs, edge_batch_idxs, edges,
           rela_embed, Ws, Wr, Wqr_w, Wqr_b, walpha_w, walpha_b, Wh,
           gru_w_ih, gru_w_hh, gru_b_ih, gru_b_hh, W_final):
    N = batch_idxs.shape[0]
    Ln, R, H = rela_embed.shape
    A = Ws.shape[1]
    E = edges.shape[0]
    MB = 512
    NP = 10240
    NBLK = NP // MB

    sub = edges[:, 0].astype(jnp.int32)
    rel = edges[:, 1].astype(jnp.int32)
    obj = edges[:, 2].astype(jnp.int32)
    eb = edge_batch_idxs.astype(jnp.int32)
    EW = E // _NW
    NCHUNK = EW // _K
    idx_pack = (jnp.stack([sub, rel, obj, eb], 0)
                .reshape(4, _NW, NCHUNK, _K)
                .transpose(1, 2, 0, 3)
                .reshape(_NW * NCHUNK, 4, _K))

    h = jnp.zeros((NP, H), jnp.float32).at[query_sub_idxs].set(1.0)
    h0ext = jnp.zeros((NP, 2 * _HE), jnp.float32)

    rela_pad = jnp.pad(rela_embed, ((0, 0), (0, _RPAD - R), (0, 0)))
    re_q = jnp.pad(rela_embed[:, q_rel, :],
                   ((0, 0), (0, _BPAD - q_rel.shape[0]), (0, 0)))
    Ws_p = jnp.pad(Ws, ((0, 0), (0, _APAD - A), (0, 0)))
    Wr_p = jnp.pad(Wr, ((0, 0), (0, _APAD - A), (0, 0)))
    Wq_p = jnp.pad(Wqr_w, ((0, 0), (0, _APAD - A), (0, 0)))
    Wf_p = jnp.pad(W_final, ((0, _APAD - 1), (0, 0)))
    bq = jnp.broadcast_to(
        jnp.pad(Wqr_b, ((0, 0), (0, _APAD - A)))[:, :, None], (Ln, _APAD, _BPAD))
    wal = jnp.concatenate(
        [walpha_w[:, 0, :], walpha_b,
         jnp.zeros((Ln, _LANES - A - 1), jnp.float32)], axis=1)
    bih2 = jnp.broadcast_to(gru_b_ih[None, :], (8, 3 * H))
    bhh2 = jnp.broadcast_to(gru_b_hh[None, :], (8, 3 * H))

    rextA, rextB, aqT = pl.pallas_call(
        _prep_tables_kernel,
        out_shape=[jax.ShapeDtypeStruct((Ln, _RPAD, _HE), jnp.float32),
                   jax.ShapeDtypeStruct((Ln, _RPAD, _HE), jnp.float32),
                   jax.ShapeDtypeStruct((Ln, _APAD, _BPAD), jnp.float32)],
    )(rela_pad, Wr_p, re_q, Wq_p, bq)

    hextA, hextB = pl.pallas_call(
        _hext0_kernel,
        grid=(NBLK,),
        in_specs=[pl.BlockSpec((MB, H), lambda j: (j, 0)),
                  pl.BlockSpec((_APAD, H), lambda j: (0, 0))],
        out_specs=[pl.BlockSpec((MB, _HE), lambda j: (j, 0)),
                   pl.BlockSpec((MB, _HE), lambda j: (j, 0))],
        out_shape=[jax.ShapeDtypeStruct((NP, _HE), jnp.float32),
                   jax.ShapeDtypeStruct((NP, _HE), jnp.float32)],
    )(h, Ws_p[0])

    sc_edge = _make_sc_edge(NP, E)

    dense_call = pl.pallas_call(
        _dense_layer_kernel,
        grid=(NBLK,),
        in_specs=[pl.BlockSpec((MB, _HH), lambda j: (j, 0)),
                  pl.BlockSpec((MB, _HH), lambda j: (j + NBLK, 0)),
                  pl.BlockSpec((MB, _HH), lambda j: (j, 0)),
                  pl.BlockSpec((MB, _HH), lambda j: (j + NBLK, 0)),
                  pl.BlockSpec((MB, 2 * _HE), lambda j: (j, 0)),
                  pl.BlockSpec((H, H), lambda j: (0, 0)),
                  pl.BlockSpec((3 * H, H), lambda j: (0, 0)),
                  pl.BlockSpec((3 * H, H), lambda j: (0, 0)),
                  pl.BlockSpec((8, 3 * H), lambda j: (0, 0)),
                  pl.BlockSpec((8, 3 * H), lambda j: (0, 0)),
                  pl.BlockSpec((_APAD, H), lambda j: (0, 0))],
        out_specs=[pl.BlockSpec((MB, _HE), lambda j: (j, 0)),
                   pl.BlockSpec((MB, _HE), lambda j: (j, 0))],
        out_shape=[jax.ShapeDtypeStruct((NP, _HE), jnp.float32),
                   jax.ShapeDtypeStruct((NP, _HE), jnp.float32)],
    )

    for i in range(Ln):
        aggA = sc_edge(hextA, rextA[i], aqT[i], wal[i], idx_pack)
        aggB = sc_edge(hextB, rextB[i], aqT[i], wal[i], idx_pack)
        wnext = Ws_p[i + 1] if i + 1 < Ln else Wf_p
        hextA, hextB = dense_call(aggA, aggA, aggB, aggB, h0ext, Wh[i],
                                  gru_w_ih, gru_w_hh, bih2, bhh2, wnext)
        h0ext = jnp.concatenate([hextA, hextB], axis=1)

    return hextA[:N, _HH]


# speedup vs baseline: 1.0216x; 1.0001x over previous
import functools

import jax
import jax.numpy as jnp
from jax import lax
from jax.experimental import pallas as pl
from jax.experimental.pallas import tpu as pltpu
from jax.experimental.pallas import tpu_sc as plsc

_NC = 2
_NS = 16
_NW = _NC * _NS
_LANES = 16

_K = 80
_RPAD = 480
_BPAD = 64
_APAD = 8
_HH = 64
_HE = 80


def _prep_tables_kernel(re_ref, wr_ref, req_ref, wq_ref, bq_ref,
                        ra_ref, rb_ref, aq_ref):
    Ln = re_ref.shape[0]
    R = re_ref.shape[1]
    z = jnp.zeros((R, _APAD), jnp.float32)
    for i in range(Ln):
        re_i = re_ref[i]
        ar = lax.dot_general(re_i, wr_ref[i], (((1,), (1,)), ((), ())),
                             preferred_element_type=jnp.float32)
        ra_ref[i] = jnp.concatenate([re_i[:, :_HH], ar, z], axis=1)
        rb_ref[i] = jnp.concatenate([re_i[:, _HH:], ar, z], axis=1)
        aq_ref[i] = lax.dot_general(
            wq_ref[i], req_ref[i], (((1,), (1,)), ((), ())),
            preferred_element_type=jnp.float32) + bq_ref[i]


def _hext0_kernel(h_ref, ws_ref, outa_ref, outb_ref):
    h = h_ref[...]
    at = lax.dot_general(h, ws_ref[...], (((1,), (1,)), ((), ())),
                         preferred_element_type=jnp.float32)
    z = jnp.zeros((h.shape[0], _APAD), jnp.float32)
    outa_ref[...] = jnp.concatenate([h[:, :_HH], at, z], axis=1)
    outb_ref[...] = jnp.concatenate([h[:, _HH:], at, z], axis=1)


def _dense_layer_kernel(a0_ref, a1_ref, b0_ref, b1_ref, h0_ref, wh_ref,
                        wih_ref, whh_ref, bih_ref, bhh_ref, wn_ref,
                        outa_ref, outb_ref):
    agg = jnp.concatenate([a0_ref[...] + a1_ref[...],
                           b0_ref[...] + b1_ref[...]], axis=1)
    hid = jnp.maximum(
        lax.dot_general(agg, wh_ref[...], (((1,), (1,)), ((), ())),
                        preferred_element_type=jnp.float32), 0.0)
    act0 = (jnp.sum(hid, axis=1, keepdims=True) == 0.0)
    mask = jnp.where(act0, 0.0, 1.0)
    Hd = hid.shape[1]
    h0 = jnp.concatenate([h0_ref[:, :_HH], h0_ref[:, _HE:_HE + _HH]], axis=1)
    gi = lax.dot_general(hid, wih_ref[...], (((1,), (1,)), ((), ())),
                         preferred_element_type=jnp.float32) + bih_ref[0:1, :]
    gh = lax.dot_general(h0, whh_ref[...], (((1,), (1,)), ((), ())),
                         preferred_element_type=jnp.float32) + bhh_ref[0:1, :]
    i_r, i_z, i_n = gi[:, :Hd], gi[:, Hd:2 * Hd], gi[:, 2 * Hd:]
    h_r, h_z, h_n = gh[:, :Hd], gh[:, Hd:2 * Hd], gh[:, 2 * Hd:]
    r = jax.nn.sigmoid(i_r + h_r)
    z = jax.nn.sigmoid(i_z + h_z)
    n = jnp.tanh(i_n + r * h_n)
    hnew = (1.0 - z) * n + z * h0
    st = hnew * mask
    at = lax.dot_general(st, wn_ref[...], (((1,), (1,)), ((), ())),
                         preferred_element_type=jnp.float32)
    zp = jnp.zeros((st.shape[0], _APAD), jnp.float32)
    outa_ref[...] = jnp.concatenate([st[:, :_HH], at, zp], axis=1)
    outb_ref[...] = jnp.concatenate([st[:, _HH:], at, zp], axis=1)


def _make_sc_edge(NP, E):
    EW = E // _NW
    NCHUNK = EW // _K
    NR = NP // _NS
    RC = _K
    NRC = NR // RC
    AOF = _HH
    mesh = plsc.VectorSubcoreMesh(core_axis_name="c", subcore_axis_name="s",
                                  num_cores=_NC, num_subcores=_NS)

    @functools.partial(
        pl.kernel,
        out_type=jax.ShapeDtypeStruct((_NC * NP, _HH), jnp.float32),
        mesh=mesh,
        compiler_params=pltpu.CompilerParams(use_tc_tiling_on_sc=False,
                                             needs_layout_passes=False),
        scratch_types=[
            pltpu.VMEM_SHARED((NP, _HH), jnp.float32),
            pltpu.VMEM((5 * _BPAD,), jnp.float32),
            pltpu.VMEM((_LANES,), jnp.float32),
            pltpu.VMEM((2, 4, _K), jnp.int32),
            pltpu.VMEM((2, _K, _HE), jnp.float32),
            pltpu.VMEM((2, _K, _HE), jnp.float32),
            pltpu.VMEM((2, _K, _HH), jnp.float32),
            pltpu.VMEM((2, _K), jnp.int32),
            pltpu.VMEM((_K,), jnp.float32),
            pltpu.SemaphoreType.DMA,
            pltpu.SemaphoreType.DMA,
            pltpu.SemaphoreType.DMA,
            pltpu.SemaphoreType.DMA,
            pltpu.SemaphoreType.DMA,
            pltpu.SemaphoreType.DMA,
        ],
    )
    def sc_edge(hext_hbm, rext_hbm, aq_hbm, wal_hbm,
                idx_hbm, out_hbm,
                agg_sh, cols_q, wbuf,
                idx_c, hext_buf, rext_buf, msg_buf, obj_g, alpha_buf,
                semh0, semh1, semr0, semr1, sems0, sems1):
        cid = lax.axis_index("c")
        sid = lax.axis_index("s")
        wid = sid * _NC + cid
        semh = (semh0, semh1)
        semr = (semr0, semr1)
        sems = (sems0, sems1)

        for a in range(5):
            pltpu.sync_copy(aq_hbm.at[a], cols_q.at[pl.ds(a * _BPAD, _BPAD)])
        pltpu.sync_copy(wal_hbm, wbuf)

        bounce = msg_buf.at[0]
        def zrow(rr, _):
            for j in range(_HH // _LANES):
                bounce[rr, pl.ds(_LANES * j, _LANES)] = jnp.zeros(
                    (_LANES,), jnp.float32)
            return 0
        lax.fori_loop(0, RC, zrow, 0)
        for t in range(NRC):
            pltpu.sync_copy(bounce, agg_sh.at[pl.ds(sid * NR + t * RC, RC)])
        plsc.subcore_barrier()

        blk0 = wid * NCHUNK
        wv = wbuf[...]
        lanes = jnp.arange(_LANES, dtype=jnp.int32)

        def fetch(blk, s):
            pltpu.sync_copy(idx_hbm.at[blk], idx_c.at[s])
            pltpu.async_copy(hext_hbm.at[idx_c.at[s].at[0]],
                             hext_buf.at[s], semh[s])
            pltpu.async_copy(rext_hbm.at[idx_c.at[s].at[1]],
                             rext_buf.at[s], semr[s])

        def compute(blk, s, wait_pred, fetch_pred):
            hb = hext_buf.at[s]
            rb = rext_buf.at[s]
            ms = msg_buf.at[s]
            og = obj_g.at[s]
            ic = idx_c.at[s]
            pltpu.make_async_copy(hext_hbm.at[ic.at[0]], hb, semh[s]).wait()
            pltpu.make_async_copy(rext_hbm.at[ic.at[1]], rb, semr[s]).wait()

            @plsc.parallel_loop(0, _K // _LANES, step=1,
                                unroll=_K // _LANES)
            def _(g):
                sl = pl.ds(_LANES * g, _LANES)
                e16 = lanes + (_LANES * g)
                og[sl] = ic[2, sl]
                eb16 = ic[3, sl]
                acc = jnp.full((_LANES,), wv[5])
                for a in range(5):
                    ca = jnp.full((_LANES,), AOF + a, jnp.int32)
                    va = (plsc.load_gather(hb, [e16, ca])
                          + plsc.load_gather(rb, [e16, ca])
                          + plsc.load_gather(cols_q, [eb16 + a * _BPAD]))
                    va = jnp.maximum(va, 0.0)
                    acc = acc + jnp.full((_LANES,), wv[a]) * va
                alpha_buf[sl] = 1.0 / (1.0 + jnp.exp(-acc))

            @pl.when(wait_pred)
            def _():
                pltpu.make_async_copy(ms, agg_sh.at[og], sems[s]).wait()

            @plsc.parallel_loop(0, _K, step=1, unroll=10)
            def _(e):
                av = plsc.load_gather(
                    alpha_buf, [jnp.full((_LANES,), e, jnp.int32)])
                for j in range(_HH // _LANES):
                    s2 = pl.ds(_LANES * j, _LANES)
                    ms[e, s2] = hb[e, s2] * rb[e, s2] * av

            pltpu.async_copy(ms, agg_sh.at[og], sems[s], add=True)

            @pl.when(fetch_pred)
            def _():
                fetch(blk + 2, s)

        false_p = jnp.bool_(False)
        true_p = jnp.bool_(True)

        fetch(blk0, 0)
        fetch(blk0 + 1, 1)

        def pair_body(p, _):
            c0 = blk0 + 2 * p
            compute(c0, 0, p > 0, true_p)
            compute(c0 + 1, 1, p > 0, (2 * p + 3) < NCHUNK)
            return 0
        lax.fori_loop(0, (NCHUNK - 1) // 2, pair_body, 0)
        compute(blk0 + NCHUNK - 1, 0, true_p, false_p)

        pltpu.make_async_copy(msg_buf.at[0], agg_sh.at[obj_g.at[0]],
                              sems0).wait()
        pltpu.make_async_copy(msg_buf.at[1], agg_sh.at[obj_g.at[1]],
                              sems1).wait()
        plsc.subcore_barrier()

        for t in range(NRC):
            row = sid * NR + t * RC
            pltpu.sync_copy(agg_sh.at[pl.ds(row, RC)], bounce)
            pltpu.sync_copy(bounce, out_hbm.at[pl.ds(cid * NP + row, RC)])

    return sc_edge


def kernel(q_sub, q_rel, batch_idxs, query_sub_idxs, edge_batch_idxs, edges,
           rela_embed, Ws, Wr, Wqr_w, Wqr_b, walpha_w, walpha_b, Wh,
           gru_w_ih, gru_w_hh, gru_b_ih, gru_b_hh, W_final):
    N = batch_idxs.shape[0]
    Ln, R, H = rela_embed.shape
    A = Ws.shape[1]
    E = edges.shape[0]
    MB = 512
    NP = 10240
    NBLK = NP // MB

    sub = edges[:, 0].astype(jnp.int32)
    rel = edges[:, 1].astype(jnp.int32)
    obj = edges[:, 2].astype(jnp.int32)
    eb = edge_batch_idxs.astype(jnp.int32)
    EW = E // _NW
    NCHUNK = EW // _K
    idx_pack = (jnp.stack([sub, rel, obj, eb], 0)
                .reshape(4, _NW, NCHUNK, _K)
                .transpose(1, 2, 0, 3)
                .reshape(_NW * NCHUNK, 4, _K))

    h = jnp.zeros((NP, H), jnp.float32).at[query_sub_idxs].set(1.0)
    h0ext = jnp.zeros((NP, 2 * _HE), jnp.float32)

    rela_pad = jnp.pad(rela_embed, ((0, 0), (0, _RPAD - R), (0, 0)))
    re_q = jnp.pad(rela_embed[:, q_rel, :],
                   ((0, 0), (0, _BPAD - q_rel.shape[0]), (0, 0)))
    Ws_p = jnp.pad(Ws, ((0, 0), (0, _APAD - A), (0, 0)))
    Wr_p = jnp.pad(Wr, ((0, 0), (0, _APAD - A), (0, 0)))
    Wq_p = jnp.pad(Wqr_w, ((0, 0), (0, _APAD - A), (0, 0)))
    Wf_p = jnp.pad(W_final, ((0, _APAD - 1), (0, 0)))
    bq = jnp.broadcast_to(
        jnp.pad(Wqr_b, ((0, 0), (0, _APAD - A)))[:, :, None], (Ln, _APAD, _BPAD))
    wal = jnp.concatenate(
        [walpha_w[:, 0, :], walpha_b,
         jnp.zeros((Ln, _LANES - A - 1), jnp.float32)], axis=1)
    bih2 = jnp.broadcast_to(gru_b_ih[None, :], (8, 3 * H))
    bhh2 = jnp.broadcast_to(gru_b_hh[None, :], (8, 3 * H))

    rextA, rextB, aqT = pl.pallas_call(
        _prep_tables_kernel,
        out_shape=[jax.ShapeDtypeStruct((Ln, _RPAD, _HE), jnp.float32),
                   jax.ShapeDtypeStruct((Ln, _RPAD, _HE), jnp.float32),
                   jax.ShapeDtypeStruct((Ln, _APAD, _BPAD), jnp.float32)],
    )(rela_pad, Wr_p, re_q, Wq_p, bq)

    hextA, hextB = pl.pallas_call(
        _hext0_kernel,
        grid=(NBLK,),
        in_specs=[pl.BlockSpec((MB, H), lambda j: (j, 0)),
                  pl.BlockSpec((_APAD, H), lambda j: (0, 0))],
        out_specs=[pl.BlockSpec((MB, _HE), lambda j: (j, 0)),
                   pl.BlockSpec((MB, _HE), lambda j: (j, 0))],
        out_shape=[jax.ShapeDtypeStruct((NP, _HE), jnp.float32),
                   jax.ShapeDtypeStruct((NP, _HE), jnp.float32)],
    )(h, Ws_p[0])

    sc_edge = _make_sc_edge(NP, E)

    dense_call = pl.pallas_call(
        _dense_layer_kernel,
        grid=(NBLK,),
        in_specs=[pl.BlockSpec((MB, _HH), lambda j: (j, 0)),
                  pl.BlockSpec((MB, _HH), lambda j: (j + NBLK, 0)),
                  pl.BlockSpec((MB, _HH), lambda j: (j, 0)),
                  pl.BlockSpec((MB, _HH), lambda j: (j + NBLK, 0)),
                  pl.BlockSpec((MB, 2 * _HE), lambda j: (j, 0)),
                  pl.BlockSpec((H, H), lambda j: (0, 0)),
                  pl.BlockSpec((3 * H, H), lambda j: (0, 0)),
                  pl.BlockSpec((3 * H, H), lambda j: (0, 0)),
                  pl.BlockSpec((8, 3 * H), lambda j: (0, 0)),
                  pl.BlockSpec((8, 3 * H), lambda j: (0, 0)),
                  pl.BlockSpec((_APAD, H), lambda j: (0, 0))],
        out_specs=[pl.BlockSpec((MB, _HE), lambda j: (j, 0)),
                   pl.BlockSpec((MB, _HE), lambda j: (j, 0))],
        out_shape=[jax.ShapeDtypeStruct((NP, _HE), jnp.float32),
                   jax.ShapeDtypeStruct((NP, _HE), jnp.float32)],
    )

    for i in range(Ln):
        aggA = sc_edge(hextA, rextA[i], aqT[i], wal[i], idx_pack)
        aggB = sc_edge(hextB, rextB[i], aqT[i], wal[i], idx_pack)
        wnext = Ws_p[i + 1] if i + 1 < Ln else Wf_p
        hextA, hextB = dense_call(aggA, aggA, aggB, aggB, h0ext, Wh[i],
                                  gru_w_ih, gru_w_hh, bih2, bhh2, wnext)
        h0ext = jnp.concatenate([hextA, hextB], axis=1)

    return hextA[:N, _HH]


# speedup vs baseline: 1.0227x; 1.0011x over previous
import functools

import jax
import jax.numpy as jnp
from jax import lax
from jax.experimental import pallas as pl
from jax.experimental.pallas import tpu as pltpu
from jax.experimental.pallas import tpu_sc as plsc

_NC = 2
_NS = 16
_NW = _NC * _NS
_LANES = 16

_K = 80
_RPAD = 480
_BPAD = 64
_APAD = 8
_HH = 64
_HE = 80


def _prep_tables_kernel(re_ref, wr_ref, req_ref, wq_ref, bq_ref,
                        ra_ref, rb_ref, aq_ref):
    Ln = re_ref.shape[0]
    R = re_ref.shape[1]
    z = jnp.zeros((R, _APAD), jnp.float32)
    for i in range(Ln):
        re_i = re_ref[i]
        ar = lax.dot_general(re_i, wr_ref[i], (((1,), (1,)), ((), ())),
                             preferred_element_type=jnp.float32)
        ra_ref[i] = jnp.concatenate([re_i[:, :_HH], ar, z], axis=1)
        rb_ref[i] = jnp.concatenate([re_i[:, _HH:], ar, z], axis=1)
        aq_ref[i] = lax.dot_general(
            wq_ref[i], req_ref[i], (((1,), (1,)), ((), ())),
            preferred_element_type=jnp.float32) + bq_ref[i]


def _hext0_kernel(h_ref, ws_ref, outa_ref, outb_ref):
    h = h_ref[...]
    at = lax.dot_general(h, ws_ref[...], (((1,), (1,)), ((), ())),
                         preferred_element_type=jnp.float32)
    z = jnp.zeros((h.shape[0], _APAD), jnp.float32)
    outa_ref[...] = jnp.concatenate([h[:, :_HH], at, z], axis=1)
    outb_ref[...] = jnp.concatenate([h[:, _HH:], at, z], axis=1)


def _dense_layer_kernel(a0_ref, a1_ref, b0_ref, b1_ref, h0_ref, wh_ref,
                        wih_ref, whh_ref, bih_ref, bhh_ref, wn_ref,
                        outa_ref, outb_ref):
    agg = jnp.concatenate([a0_ref[...] + a1_ref[...],
                           b0_ref[...] + b1_ref[...]], axis=1)
    hid = jnp.maximum(
        lax.dot_general(agg, wh_ref[...], (((1,), (1,)), ((), ())),
                        preferred_element_type=jnp.float32), 0.0)
    act0 = (jnp.sum(hid, axis=1, keepdims=True) == 0.0)
    mask = jnp.where(act0, 0.0, 1.0)
    Hd = hid.shape[1]
    h0 = jnp.concatenate([h0_ref[:, :_HH], h0_ref[:, _HE:_HE + _HH]], axis=1)
    gi = lax.dot_general(hid, wih_ref[...], (((1,), (1,)), ((), ())),
                         preferred_element_type=jnp.float32) + bih_ref[0:1, :]
    gh = lax.dot_general(h0, whh_ref[...], (((1,), (1,)), ((), ())),
                         preferred_element_type=jnp.float32) + bhh_ref[0:1, :]
    i_r, i_z, i_n = gi[:, :Hd], gi[:, Hd:2 * Hd], gi[:, 2 * Hd:]
    h_r, h_z, h_n = gh[:, :Hd], gh[:, Hd:2 * Hd], gh[:, 2 * Hd:]
    r = jax.nn.sigmoid(i_r + h_r)
    z = jax.nn.sigmoid(i_z + h_z)
    n = jnp.tanh(i_n + r * h_n)
    hnew = (1.0 - z) * n + z * h0
    st = hnew * mask
    at = lax.dot_general(st, wn_ref[...], (((1,), (1,)), ((), ())),
                         preferred_element_type=jnp.float32)
    zp = jnp.zeros((st.shape[0], _APAD), jnp.float32)
    outa_ref[...] = jnp.concatenate([st[:, :_HH], at, zp], axis=1)
    outb_ref[...] = jnp.concatenate([st[:, _HH:], at, zp], axis=1)


def _make_sc_edge(NP, E):
    EW = E // _NW
    NCHUNK = EW // _K
    NR = NP // _NS
    RC = _K
    NRC = NR // RC
    AOF = _HH
    mesh = plsc.VectorSubcoreMesh(core_axis_name="c", subcore_axis_name="s",
                                  num_cores=_NC, num_subcores=_NS)

    @functools.partial(
        pl.kernel,
        out_type=jax.ShapeDtypeStruct((_NC * NP, _HH), jnp.float32),
        mesh=mesh,
        compiler_params=pltpu.CompilerParams(use_tc_tiling_on_sc=False,
                                             needs_layout_passes=False),
        scratch_types=[
            pltpu.VMEM_SHARED((NP, _HH), jnp.float32),
            pltpu.VMEM((5 * _BPAD,), jnp.float32),
            pltpu.VMEM((_LANES,), jnp.float32),
            pltpu.VMEM((2, 4, _K), jnp.int32),
            pltpu.VMEM((2, _K, _HE), jnp.float32),
            pltpu.VMEM((2, _K, _HE), jnp.float32),
            pltpu.VMEM((2, _K, _HH), jnp.float32),
            pltpu.VMEM((2, _K), jnp.int32),
            pltpu.VMEM((_K,), jnp.float32),
            pltpu.SemaphoreType.DMA,
            pltpu.SemaphoreType.DMA,
            pltpu.SemaphoreType.DMA,
            pltpu.SemaphoreType.DMA,
            pltpu.SemaphoreType.DMA,
            pltpu.SemaphoreType.DMA,
        ],
    )
    def sc_edge(hext_hbm, rext_hbm, aq_hbm, wal_hbm,
                idx_hbm, out_hbm,
                agg_sh, cols_q, wbuf,
                idx_c, hext_buf, rext_buf, msg_buf, obj_g, alpha_buf,
                semh0, semh1, semr0, semr1, sems0, sems1):
        cid = lax.axis_index("c")
        sid = lax.axis_index("s")
        wid = sid * _NC + cid
        semh = (semh0, semh1)
        semr = (semr0, semr1)
        sems = (sems0, sems1)

        for a in range(5):
            pltpu.sync_copy(aq_hbm.at[a], cols_q.at[pl.ds(a * _BPAD, _BPAD)])
        pltpu.sync_copy(wal_hbm, wbuf)

        bounce = msg_buf.at[0]
        def zrow(rr, _):
            for j in range(_HH // _LANES):
                bounce[rr, pl.ds(_LANES * j, _LANES)] = jnp.zeros(
                    (_LANES,), jnp.float32)
            return 0
        lax.fori_loop(0, RC, zrow, 0)
        for t in range(NRC):
            pltpu.sync_copy(bounce, agg_sh.at[pl.ds(sid * NR + t * RC, RC)])
        plsc.subcore_barrier()

        blk0 = wid * NCHUNK
        wv = wbuf[...]
        lanes = jnp.arange(_LANES, dtype=jnp.int32)

        def fetch(blk, s):
            pltpu.sync_copy(idx_hbm.at[blk], idx_c.at[s])
            pltpu.async_copy(hext_hbm.at[idx_c.at[s].at[0]],
                             hext_buf.at[s], semh[s])
            pltpu.async_copy(rext_hbm.at[idx_c.at[s].at[1]],
                             rext_buf.at[s], semr[s])

        def compute(blk, s, wait_pred, fetch_pred):
            hb = hext_buf.at[s]
            rb = rext_buf.at[s]
            ms = msg_buf.at[s]
            og = obj_g.at[s]
            ic = idx_c.at[s]
            pltpu.make_async_copy(hext_hbm.at[ic.at[0]], hb, semh[s]).wait()
            pltpu.make_async_copy(rext_hbm.at[ic.at[1]], rb, semr[s]).wait()

            @plsc.parallel_loop(0, _K // _LANES, step=1,
                                unroll=_K // _LANES)
            def _(g):
                sl = pl.ds(_LANES * g, _LANES)
                e16 = lanes + (_LANES * g)
                og[sl] = ic[2, sl]
                eb16 = ic[3, sl]
                acc = jnp.full((_LANES,), wv[5])
                for a in range(5):
                    ca = jnp.full((_LANES,), AOF + a, jnp.int32)
                    va = (plsc.load_gather(hb, [e16, ca])
                          + plsc.load_gather(rb, [e16, ca])
                          + plsc.load_gather(cols_q, [eb16 + a * _BPAD]))
                    va = jnp.maximum(va, 0.0)
                    acc = acc + jnp.full((_LANES,), wv[a]) * va
                alpha_buf[sl] = 1.0 / (1.0 + jnp.exp(-acc))

            @pl.when(wait_pred)
            def _():
                pltpu.make_async_copy(ms, agg_sh.at[og], sems[s]).wait()

            @plsc.parallel_loop(0, _K, step=1, unroll=8)
            def _(e):
                av = plsc.load_gather(
                    alpha_buf, [jnp.full((_LANES,), e, jnp.int32)])
                for j in range(_HH // _LANES):
                    s2 = pl.ds(_LANES * j, _LANES)
                    ms[e, s2] = hb[e, s2] * rb[e, s2] * av

            pltpu.async_copy(ms, agg_sh.at[og], sems[s], add=True)

            @pl.when(fetch_pred)
            def _():
                fetch(blk + 2, s)

        false_p = jnp.bool_(False)
        true_p = jnp.bool_(True)

        fetch(blk0, 0)
        fetch(blk0 + 1, 1)

        def pair_body(p, _):
            c0 = blk0 + 2 * p
            compute(c0, 0, p > 0, true_p)
            compute(c0 + 1, 1, p > 0, (2 * p + 3) < NCHUNK)
            return 0
        lax.fori_loop(0, (NCHUNK - 1) // 2, pair_body, 0)
        compute(blk0 + NCHUNK - 1, 0, true_p, false_p)

        pltpu.make_async_copy(msg_buf.at[0], agg_sh.at[obj_g.at[0]],
                              sems0).wait()
        pltpu.make_async_copy(msg_buf.at[1], agg_sh.at[obj_g.at[1]],
                              sems1).wait()
        plsc.subcore_barrier()

        for t in range(NRC):
            row = sid * NR + t * RC
            pltpu.sync_copy(agg_sh.at[pl.ds(row, RC)], bounce)
            pltpu.sync_copy(bounce, out_hbm.at[pl.ds(cid * NP + row, RC)])

    return sc_edge


def kernel(q_sub, q_rel, batch_idxs, query_sub_idxs, edge_batch_idxs, edges,
           rela_embed, Ws, Wr, Wqr_w, Wqr_b, walpha_w, walpha_b, Wh,
           gru_w_ih, gru_w_hh, gru_b_ih, gru_b_hh, W_final):
    N = batch_idxs.shape[0]
    Ln, R, H = rela_embed.shape
    A = Ws.shape[1]
    E = edges.shape[0]
    MB = 512
    NP = 10240
    NBLK = NP // MB

    sub = edges[:, 0].astype(jnp.int32)
    rel = edges[:, 1].astype(jnp.int32)
    obj = edges[:, 2].astype(jnp.int32)
    eb = edge_batch_idxs.astype(jnp.int32)
    EW = E // _NW
    NCHUNK = EW // _K
    idx_pack = (jnp.stack([sub, rel, obj, eb], 0)
                .reshape(4, _NW, NCHUNK, _K)
                .transpose(1, 2, 0, 3)
                .reshape(_NW * NCHUNK, 4, _K))

    h = jnp.zeros((NP, H), jnp.float32).at[query_sub_idxs].set(1.0)
    h0ext = jnp.zeros((NP, 2 * _HE), jnp.float32)

    rela_pad = jnp.pad(rela_embed, ((0, 0), (0, _RPAD - R), (0, 0)))
    re_q = jnp.pad(rela_embed[:, q_rel, :],
                   ((0, 0), (0, _BPAD - q_rel.shape[0]), (0, 0)))
    Ws_p = jnp.pad(Ws, ((0, 0), (0, _APAD - A), (0, 0)))
    Wr_p = jnp.pad(Wr, ((0, 0), (0, _APAD - A), (0, 0)))
    Wq_p = jnp.pad(Wqr_w, ((0, 0), (0, _APAD - A), (0, 0)))
    Wf_p = jnp.pad(W_final, ((0, _APAD - 1), (0, 0)))
    bq = jnp.broadcast_to(
        jnp.pad(Wqr_b, ((0, 0), (0, _APAD - A)))[:, :, None], (Ln, _APAD, _BPAD))
    wal = jnp.concatenate(
        [walpha_w[:, 0, :], walpha_b,
         jnp.zeros((Ln, _LANES - A - 1), jnp.float32)], axis=1)
    bih2 = jnp.broadcast_to(gru_b_ih[None, :], (8, 3 * H))
    bhh2 = jnp.broadcast_to(gru_b_hh[None, :], (8, 3 * H))

    rextA, rextB, aqT = pl.pallas_call(
        _prep_tables_kernel,
        out_shape=[jax.ShapeDtypeStruct((Ln, _RPAD, _HE), jnp.float32),
                   jax.ShapeDtypeStruct((Ln, _RPAD, _HE), jnp.float32),
                   jax.ShapeDtypeStruct((Ln, _APAD, _BPAD), jnp.float32)],
    )(rela_pad, Wr_p, re_q, Wq_p, bq)

    hextA, hextB = pl.pallas_call(
        _hext0_kernel,
        grid=(NBLK,),
        in_specs=[pl.BlockSpec((MB, H), lambda j: (j, 0)),
                  pl.BlockSpec((_APAD, H), lambda j: (0, 0))],
        out_specs=[pl.BlockSpec((MB, _HE), lambda j: (j, 0)),
                   pl.BlockSpec((MB, _HE), lambda j: (j, 0))],
        out_shape=[jax.ShapeDtypeStruct((NP, _HE), jnp.float32),
                   jax.ShapeDtypeStruct((NP, _HE), jnp.float32)],
    )(h, Ws_p[0])

    sc_edge = _make_sc_edge(NP, E)

    dense_call = pl.pallas_call(
        _dense_layer_kernel,
        grid=(NBLK,),
        in_specs=[pl.BlockSpec((MB, _HH), lambda j: (j, 0)),
                  pl.BlockSpec((MB, _HH), lambda j: (j + NBLK, 0)),
                  pl.BlockSpec((MB, _HH), lambda j: (j, 0)),
                  pl.BlockSpec((MB, _HH), lambda j: (j + NBLK, 0)),
                  pl.BlockSpec((MB, 2 * _HE), lambda j: (j, 0)),
                  pl.BlockSpec((H, H), lambda j: (0, 0)),
                  pl.BlockSpec((3 * H, H), lambda j: (0, 0)),
                  pl.BlockSpec((3 * H, H), lambda j: (0, 0)),
                  pl.BlockSpec((8, 3 * H), lambda j: (0, 0)),
                  pl.BlockSpec((8, 3 * H), lambda j: (0, 0)),
                  pl.BlockSpec((_APAD, H), lambda j: (0, 0))],
        out_specs=[pl.BlockSpec((MB, _HE), lambda j: (j, 0)),
                   pl.BlockSpec((MB, _HE), lambda j: (j, 0))],
        out_shape=[jax.ShapeDtypeStruct((NP, _HE), jnp.float32),
                   jax.ShapeDtypeStruct((NP, _HE), jnp.float32)],
    )

    for i in range(Ln):
        aggA = sc_edge(hextA, rextA[i], aqT[i], wal[i], idx_pack)
        aggB = sc_edge(hextB, rextB[i], aqT[i], wal[i], idx_pack)
        wnext = Ws_p[i + 1] if i + 1 < Ln else Wf_p
        hextA, hextB = dense_call(aggA, aggA, aggB, aggB, h0ext, Wh[i],
                                  gru_w_ih, gru_w_hh, bih2, bhh2, wnext)
        h0ext = jnp.concatenate([hextA, hextB], axis=1)

    return hextA[:N, _HH]
